# Initial kernel scaffold; baseline (speedup 1.0000x reference)
import jax
import jax.numpy as jnp
from jax.experimental import pallas as pl


def kernel(ents_embed_input, edge_index, edge_weight, W_ent, bias_vec):
    raise NotImplementedError("write your pallas kernel here")


# speedup vs baseline: 2.9546x; 2.9546x over previous
import functools

import jax
import jax.numpy as jnp
from jax import lax
from jax.experimental import pallas as pl
from jax.experimental.pallas import tpu as pltpu
from jax.experimental.pallas import tpu_sc as plsc

_EPS = 1e-5
_MIN_NORM = 1e-10
_K = 80
_NS = 16
_HALF = 128


def _norm(x):
    return jnp.sqrt(jnp.clip(jnp.sum(x * x, axis=-1, keepdims=True), _MIN_NORM, None))


def _atanh(x):
    return 0.5 * jnp.log((1.0 + x) / (1.0 - x))


def _log0(x):
    n = _norm(x)
    nc = jnp.clip(n, _MIN_NORM, 1.0 - _EPS)
    return _atanh(nc) * x / n


def _exp0(v):
    n = _norm(v)
    return jnp.tanh(n) * v / n


def _proj(x):
    n = _norm(x)
    scale = jnp.minimum(jnp.ones_like(n), (1.0 - _EPS) / n)
    return x * scale


def _mobius(x, y):
    x2 = jnp.sum(x * x, axis=-1, keepdims=True)
    y2 = jnp.sum(y * y, axis=-1, keepdims=True)
    xy = jnp.sum(x * y, axis=-1, keepdims=True)
    num = (1.0 + 2.0 * xy + y2) * x + (1.0 - x2) * y
    den = 1.0 + 2.0 * xy + x2 * y2
    return num / (den + _MIN_NORM)


def _head_body(x_ref, w_ref, lo_ref, hi_ref):
    t = _log0(x_ref[...])
    m = jnp.dot(t, w_ref[...], preferred_element_type=jnp.float32)
    lo_ref[...] = m[:, :_HALF]
    hi_ref[...] = m[:, _HALF:]


def _head(x, W):
    N, D = x.shape
    bn = 2000
    return pl.pallas_call(
        _head_body,
        grid=(N // bn,),
        in_specs=[
            pl.BlockSpec((bn, D), lambda i: (i, 0)),
            pl.BlockSpec((D, D), lambda i: (0, 0)),
        ],
        out_specs=[
            pl.BlockSpec((bn, _HALF), lambda i: (i, 0)),
            pl.BlockSpec((bn, _HALF), lambda i: (i, 0)),
        ],
        out_shape=[
            jax.ShapeDtypeStruct((N, _HALF), jnp.float32),
            jax.ShapeDtypeStruct((N, _HALF), jnp.float32),
        ],
    )(x, W)


@functools.lru_cache(maxsize=None)
def _make_agg(N, EP):
    NP = -(-N // (_NS * 8)) * (_NS * 8)
    stripe = NP // _NS
    chunks = EP // (_NS * _K)
    epw = chunks * _K
    mesh = plsc.VectorSubcoreMesh(core_axis_name="c", subcore_axis_name="s")

    @functools.partial(
        pl.kernel,
        mesh=mesh,
        out_type=[
            jax.ShapeDtypeStruct((NP, _HALF), jnp.float32),
            jax.ShapeDtypeStruct((NP, _HALF), jnp.float32),
        ],
        scratch_types=[
            pltpu.VMEM((_K,), jnp.int32),
            pltpu.VMEM((_K,), jnp.int32),
            pltpu.VMEM((_K,), jnp.float32),
            pltpu.VMEM((_K, _HALF), jnp.float32),
            pltpu.VMEM_SHARED((NP, _HALF), jnp.float32),
            pltpu.SemaphoreType.DMA,
        ],
    )
    def agg(lo_hbm, hi_hbm, src_hbm, dst_hbm, w_hbm, z_hbm,
            out_lo, out_hi, src_v, dst_v, w_v, rows_v, acc_sh, sem):
        c = lax.axis_index("c")
        s = lax.axis_index("s")

        def process(tab_hbm, out_hbm):
            pltpu.sync_copy(z_hbm.at[pl.ds(s * stripe, stripe)],
                            acc_sh.at[pl.ds(s * stripe, stripe)])
            plsc.subcore_barrier()
            ebase = s * epw

            def chunk(ci, carry):
                base = ebase + ci * _K
                pltpu.sync_copy(src_hbm.at[pl.ds(base, _K)], src_v)
                pltpu.sync_copy(dst_hbm.at[pl.ds(base, _K)], dst_v)
                pltpu.sync_copy(w_hbm.at[pl.ds(base, _K)], w_v)
                pltpu.async_copy(tab_hbm.at[src_v], rows_v, sem).wait()

                def scale(g, cc):
                    w16 = w_v[pl.ds(g * 16, 16)]
                    for el in range(16):
                        wv = w16[el]
                        e = g * 16 + el
                        for d in range(_HALF // 16):
                            sl = pl.ds(d * 16, 16)
                            rows_v[e, sl] = rows_v[e, sl] * wv
                    return cc

                lax.fori_loop(0, _K // 16, scale, 0)
                pltpu.sync_copy(rows_v, acc_sh.at[dst_v], add=True)
                return carry

            lax.fori_loop(0, chunks, chunk, 0)
            plsc.subcore_barrier()
            pltpu.sync_copy(acc_sh.at[pl.ds(s * stripe, stripe)],
                            out_hbm.at[pl.ds(s * stripe, stripe)])

        @pl.when(c == 0)
        def _():
            process(lo_hbm, out_lo)

        @pl.when(c == 1)
        def _():
            process(hi_hbm, out_hi)

    return agg


def _tail_body(lo_ref, hi_ref, b_ref, out_ref):
    h = jnp.concatenate([lo_ref[...], hi_ref[...]], axis=-1)
    h = _proj(_exp0(h))
    b = _proj(_exp0(b_ref[...]))
    h = _mobius(h, b)
    h = _proj(h)
    h = jnp.tanh(_log0(h))
    out_ref[...] = _proj(_exp0(h))


def _tail(lo, hi, bias_vec, N):
    D = 2 * _HALF
    bn = 2000
    return pl.pallas_call(
        _tail_body,
        grid=(N // bn,),
        in_specs=[
            pl.BlockSpec((bn, _HALF), lambda i: (i, 0)),
            pl.BlockSpec((bn, _HALF), lambda i: (i, 0)),
            pl.BlockSpec((1, D), lambda i: (0, 0)),
        ],
        out_specs=pl.BlockSpec((bn, D), lambda i: (i, 0)),
        out_shape=jax.ShapeDtypeStruct((N, D), jnp.float32),
    )(lo, hi, bias_vec)


def kernel(ents_embed_input, edge_index, edge_weight, W_ent, bias_vec):
    N, D = ents_embed_input.shape
    E = edge_index.shape[1]
    src = edge_index[0].astype(jnp.int32)
    dst = edge_index[1].astype(jnp.int32)
    w = edge_weight.astype(jnp.float32)

    EP = -(-E // (_NS * _K)) * (_NS * _K)
    if EP != E:
        src = jnp.pad(src, (0, EP - E))
        dst = jnp.pad(dst, (0, EP - E))
        w = jnp.pad(w, (0, EP - E))

    lo, hi = _head(ents_embed_input, W_ent)
    NP = -(-N // (_NS * 8)) * (_NS * 8)
    zeros = jnp.zeros((NP, _HALF), jnp.float32)
    agg_lo, agg_hi = _make_agg(N, EP)(lo, hi, src, dst, w, zeros)
    return _tail(agg_lo, agg_hi, bias_vec, N)


# speedup vs baseline: 3.7753x; 1.2778x over previous
import functools

import jax
import jax.numpy as jnp
from jax import lax
from jax.experimental import pallas as pl
from jax.experimental.pallas import tpu as pltpu
from jax.experimental.pallas import tpu_sc as plsc

_EPS = 1e-5
_MIN_NORM = 1e-10
_K = 64
_NS = 16
_HALF = 128


def _norm(x):
    return jnp.sqrt(jnp.clip(jnp.sum(x * x, axis=-1, keepdims=True), _MIN_NORM, None))


def _atanh(x):
    return 0.5 * jnp.log((1.0 + x) / (1.0 - x))


def _log0(x):
    n = _norm(x)
    nc = jnp.clip(n, _MIN_NORM, 1.0 - _EPS)
    return _atanh(nc) * x / n


def _exp0(v):
    n = _norm(v)
    return jnp.tanh(n) * v / n


def _proj(x):
    n = _norm(x)
    scale = jnp.minimum(jnp.ones_like(n), (1.0 - _EPS) / n)
    return x * scale


def _mobius(x, y):
    x2 = jnp.sum(x * x, axis=-1, keepdims=True)
    y2 = jnp.sum(y * y, axis=-1, keepdims=True)
    xy = jnp.sum(x * y, axis=-1, keepdims=True)
    num = (1.0 + 2.0 * xy + y2) * x + (1.0 - x2) * y
    den = 1.0 + 2.0 * xy + x2 * y2
    return num / (den + _MIN_NORM)


def _head_body(x_ref, w_ref, lo_ref, hi_ref):
    t = _log0(x_ref[...])
    m = jnp.dot(t, w_ref[...], preferred_element_type=jnp.float32)
    lo_ref[...] = m[:, :_HALF]
    hi_ref[...] = m[:, _HALF:]


def _head(x, W):
    N, D = x.shape
    bn = 2000
    return pl.pallas_call(
        _head_body,
        grid=(N // bn,),
        in_specs=[
            pl.BlockSpec((bn, D), lambda i: (i, 0)),
            pl.BlockSpec((D, D), lambda i: (0, 0)),
        ],
        out_specs=[
            pl.BlockSpec((bn, _HALF), lambda i: (i, 0)),
            pl.BlockSpec((bn, _HALF), lambda i: (i, 0)),
        ],
        out_shape=[
            jax.ShapeDtypeStruct((N, _HALF), jnp.float32),
            jax.ShapeDtypeStruct((N, _HALF), jnp.float32),
        ],
    )(x, W)


@functools.lru_cache(maxsize=None)
def _make_agg(N, CH):
    NP = -(-N // (_NS * 8)) * (_NS * 8)
    stripe = NP // _NS
    mesh = plsc.VectorSubcoreMesh(core_axis_name="c", subcore_axis_name="s")

    @functools.partial(
        pl.kernel,
        mesh=mesh,
        out_type=[
            jax.ShapeDtypeStruct((NP, _HALF), jnp.float32),
            jax.ShapeDtypeStruct((NP, _HALF), jnp.float32),
        ],
        scratch_types=[
            pltpu.VMEM((2, _K), jnp.int32),
            pltpu.VMEM((2, _K), jnp.int32),
            pltpu.VMEM((2, _K), jnp.int32),
            pltpu.VMEM((2, _K), jnp.int32),
            pltpu.VMEM((1, _K), jnp.float32),
            pltpu.VMEM((1, _K), jnp.float32),
            pltpu.VMEM((1, _K), jnp.float32),
            pltpu.VMEM((1, _K), jnp.float32),
            pltpu.VMEM((_K, _HALF), jnp.float32),
            pltpu.VMEM((_K, _HALF), jnp.float32),
            pltpu.VMEM((_K, _HALF), jnp.float32),
            pltpu.VMEM((_K, _HALF), jnp.float32),
            pltpu.VMEM_SHARED((NP, _HALF), jnp.float32),
            pltpu.SemaphoreType.DMA,
            pltpu.SemaphoreType.DMA,
            pltpu.SemaphoreType.DMA,
            pltpu.SemaphoreType.DMA,
            pltpu.SemaphoreType.DMA,
            pltpu.SemaphoreType.DMA,
            pltpu.SemaphoreType.DMA,
            pltpu.SemaphoreType.DMA,
            pltpu.SemaphoreType.DMA,
            pltpu.SemaphoreType.DMA,
            pltpu.SemaphoreType.DMA,
            pltpu.SemaphoreType.DMA,
        ],
    )
    def agg(lo_hbm, hi_hbm, ed_hbm, wt_hbm, z_hbm,
            out_lo, out_hi, i0, i1, i2, i3, w0, w1, w2, w3,
            g0, g1, s0, s1, acc_sh,
            si0, si1, si2, si3, sw0, sw1, sw2, sw3, sg0, sg1, ss0, ss1):
        c = lax.axis_index("c")
        s = lax.axis_index("s")
        ibufs = (i0, i1, i2, i3)
        isems = (si0, si1, si2, si3)
        wbufs = (w0, w1, w2, w3)
        wsems = (sw0, sw1, sw2, sw3)
        gbufs = ((g0, sg0), (g1, sg1))
        sbufs = ((s0, ss0), (s1, ss1))

        def process(tab_hbm, out_hbm):
            pltpu.sync_copy(z_hbm.at[pl.ds(s * stripe, stripe)],
                            acc_sh.at[pl.ds(s * stripe, stripe)])
            plsc.subcore_barrier()
            crow = s * CH
            for j in range(4):
                pltpu.async_copy(ed_hbm.at[crow + j], ibufs[j], isems[j])
                pltpu.async_copy(wt_hbm.at[crow + j], wbufs[j], wsems[j])
            for j in range(2):
                pltpu.make_async_copy(ed_hbm.at[crow + j], ibufs[j], isems[j]).wait()
            pltpu.async_copy(tab_hbm.at[i0.at[0]], g0, sg0)
            pltpu.async_copy(tab_hbm.at[i1.at[0]], g1, sg1)

            def quad(q, carry):
                for j in range(4):
                    ci = q * 4 + j
                    gb, sg = gbufs[j % 2]
                    sb, ss = sbufs[j % 2]
                    ib, isem = ibufs[j], isems[j]
                    wb, wsem = wbufs[j], wsems[j]
                    pltpu.make_async_copy(tab_hbm.at[ib.at[0]], gb, sg).wait()
                    pltpu.make_async_copy(wt_hbm.at[crow + ci], wb, wsem).wait()

                    @pl.when(ci >= 2)
                    def _():
                        pltpu.make_async_copy(sb, acc_sh.at[ib.at[1]], ss).wait()

                    def scale(g, cc):
                        w16 = wb[0, pl.ds(g * 16, 16)]
                        for el in range(16):
                            wv = w16[el]
                            e = g * 16 + el
                            for d in range(_HALF // 16):
                                sl = pl.ds(d * 16, 16)
                                sb[e, sl] = gb[e, sl] * wv
                        return cc

                    lax.fori_loop(0, _K // 16, scale, 0)
                    pltpu.async_copy(sb, acc_sh.at[ib.at[1]], ss, add=True)

                    @pl.when(ci + 2 < CH)
                    def _():
                        ib2, is2 = ibufs[(j + 2) % 4], isems[(j + 2) % 4]
                        pltpu.make_async_copy(
                            ed_hbm.at[crow + ci + 2], ib2, is2).wait()
                        pltpu.async_copy(tab_hbm.at[ib2.at[0]], gb, sg)

                    @pl.when(ci + 4 < CH)
                    def _():
                        pltpu.async_copy(ed_hbm.at[crow + ci + 4], ib, isem)
                        pltpu.async_copy(wt_hbm.at[crow + ci + 4], wb, wsem)
                return carry

            lax.fori_loop(0, CH // 4, quad, 0)
            pltpu.make_async_copy(s0, acc_sh.at[i2.at[1]], ss0).wait()
            pltpu.make_async_copy(s1, acc_sh.at[i3.at[1]], ss1).wait()
            plsc.subcore_barrier()
            pltpu.sync_copy(acc_sh.at[pl.ds(s * stripe, stripe)],
                            out_hbm.at[pl.ds(s * stripe, stripe)])

        @pl.when(c == 0)
        def _():
            process(lo_hbm, out_lo)

        @pl.when(c == 1)
        def _():
            process(hi_hbm, out_hi)

    return agg


def _tail_body(lo_ref, hi_ref, b_ref, out_ref):
    h = jnp.concatenate([lo_ref[...], hi_ref[...]], axis=-1)
    h = _proj(_exp0(h))
    b = _proj(_exp0(b_ref[...]))
    h = _mobius(h, b)
    h = _proj(h)
    h = jnp.tanh(_log0(h))
    out_ref[...] = _proj(_exp0(h))


def _tail(lo, hi, bias_vec, N):
    D = 2 * _HALF
    bn = 2000
    return pl.pallas_call(
        _tail_body,
        grid=(N // bn,),
        in_specs=[
            pl.BlockSpec((bn, _HALF), lambda i: (i, 0)),
            pl.BlockSpec((bn, _HALF), lambda i: (i, 0)),
            pl.BlockSpec((1, D), lambda i: (0, 0)),
        ],
        out_specs=pl.BlockSpec((bn, D), lambda i: (i, 0)),
        out_shape=jax.ShapeDtypeStruct((N, D), jnp.float32),
    )(lo, hi, bias_vec)


def kernel(ents_embed_input, edge_index, edge_weight, W_ent, bias_vec):
    N, D = ents_embed_input.shape
    E = edge_index.shape[1]
    src = edge_index[0].astype(jnp.int32)
    dst = edge_index[1].astype(jnp.int32)
    w = edge_weight.astype(jnp.float32)

    CH = -(-(-(-E // (_NS * _K))) // 8) * 8
    EP = _NS * CH * _K
    if EP != E:
        src = jnp.pad(src, (0, EP - E))
        dst = jnp.pad(dst, (0, EP - E))
        w = jnp.pad(w, (0, EP - E))
    R = _NS * CH
    ed = jnp.stack([src.reshape(R, _K), dst.reshape(R, _K)], axis=1)
    wt = w.reshape(R, 1, _K)

    lo, hi = _head(ents_embed_input, W_ent)
    NP = -(-N // (_NS * 8)) * (_NS * 8)
    zeros = jnp.zeros((NP, _HALF), jnp.float32)
    agg_lo, agg_hi = _make_agg(N, CH)(lo, hi, ed, wt, zeros)
    return _tail(agg_lo, agg_hi, bias_vec, N)


# speedup vs baseline: 5.2659x; 1.3948x over previous
import functools

import jax
import jax.numpy as jnp
from jax import lax
from jax.experimental import pallas as pl
from jax.experimental.pallas import tpu as pltpu
from jax.experimental.pallas import tpu_sc as plsc

_EPS = 1e-5
_MIN_NORM = 1e-10
_K = 96
_NS = 16
_HALF = 128


def _norm(x):
    return jnp.sqrt(jnp.clip(jnp.sum(x * x, axis=-1, keepdims=True), _MIN_NORM, None))


def _atanh(x):
    return 0.5 * jnp.log((1.0 + x) / (1.0 - x))


def _log0(x):
    n = _norm(x)
    nc = jnp.clip(n, _MIN_NORM, 1.0 - _EPS)
    return _atanh(nc) * x / n


def _exp0(v):
    n = _norm(v)
    return jnp.tanh(n) * v / n


def _proj(x):
    n = _norm(x)
    scale = jnp.minimum(jnp.ones_like(n), (1.0 - _EPS) / n)
    return x * scale


def _mobius(x, y):
    x2 = jnp.sum(x * x, axis=-1, keepdims=True)
    y2 = jnp.sum(y * y, axis=-1, keepdims=True)
    xy = jnp.sum(x * y, axis=-1, keepdims=True)
    num = (1.0 + 2.0 * xy + y2) * x + (1.0 - x2) * y
    den = 1.0 + 2.0 * xy + x2 * y2
    return num / (den + _MIN_NORM)


def _head_body(x_ref, w_ref, lo_ref, hi_ref):
    t = _log0(x_ref[...])
    m = jnp.dot(t, w_ref[...], preferred_element_type=jnp.float32)
    lo_ref[...] = m[:, :_HALF]
    hi_ref[...] = m[:, _HALF:]


def _head(x, W):
    N, D = x.shape
    bn = 2000
    return pl.pallas_call(
        _head_body,
        grid=(N // bn,),
        in_specs=[
            pl.BlockSpec((bn, D), lambda i: (i, 0)),
            pl.BlockSpec((D, D), lambda i: (0, 0)),
        ],
        out_specs=[
            pl.BlockSpec((bn, _HALF), lambda i: (i, 0)),
            pl.BlockSpec((bn, _HALF), lambda i: (i, 0)),
        ],
        out_shape=[
            jax.ShapeDtypeStruct((N, _HALF), jnp.float32),
            jax.ShapeDtypeStruct((N, _HALF), jnp.float32),
        ],
    )(x, W)


@functools.lru_cache(maxsize=None)
def _make_agg(N, CH):
    stripe = -(-(-(-N // _NS)) // 8) * 8
    NP = stripe * _NS
    mesh = plsc.VectorSubcoreMesh(core_axis_name="c", subcore_axis_name="s")

    @functools.partial(
        pl.kernel,
        mesh=mesh,
        out_type=[
            jax.ShapeDtypeStruct((NP, _HALF), jnp.float32),
            jax.ShapeDtypeStruct((NP, _HALF), jnp.float32),
        ],
        scratch_types=[
            pltpu.VMEM((2, _K), jnp.int32),
            pltpu.VMEM((2, _K), jnp.int32),
            pltpu.VMEM((2, _K), jnp.int32),
            pltpu.VMEM((2, _K), jnp.int32),
            pltpu.VMEM((1, _K), jnp.float32),
            pltpu.VMEM((1, _K), jnp.float32),
            pltpu.VMEM((1, _K), jnp.float32),
            pltpu.VMEM((1, _K), jnp.float32),
            pltpu.VMEM((_K, _HALF), jnp.float32),
            pltpu.VMEM((_K, _HALF), jnp.float32),
            pltpu.VMEM((_K, _HALF), jnp.float32),
            pltpu.VMEM_SHARED((NP, _HALF), jnp.float32),
            pltpu.SemaphoreType.DMA,
            pltpu.SemaphoreType.DMA,
            pltpu.SemaphoreType.DMA,
            pltpu.SemaphoreType.DMA,
            pltpu.SemaphoreType.DMA,
            pltpu.SemaphoreType.DMA,
            pltpu.SemaphoreType.DMA,
            pltpu.SemaphoreType.DMA,
            pltpu.SemaphoreType.DMA,
            pltpu.SemaphoreType.DMA,
            pltpu.SemaphoreType.DMA,
        ],
    )
    def agg(lo_hbm, hi_hbm, ed_hbm, wt_hbm, z_hbm,
            out_lo, out_hi, i0, i1, i2, i3, w0, w1, w2, w3,
            g0, g1, s0, acc_sh,
            si0, si1, si2, si3, sw0, sw1, sw2, sw3, sg0, sg1, ss0):
        c = lax.axis_index("c")
        s = lax.axis_index("s")
        ibufs = (i0, i1, i2, i3)
        isems = (si0, si1, si2, si3)
        wbufs = (w0, w1, w2, w3)
        wsems = (sw0, sw1, sw2, sw3)
        gbufs = ((g0, sg0), (g1, sg1))

        def process(tab_hbm, out_hbm):
            pltpu.sync_copy(z_hbm.at[pl.ds(s * stripe, stripe)],
                            acc_sh.at[pl.ds(s * stripe, stripe)])
            plsc.subcore_barrier()
            crow = s * CH
            for j in range(4):
                pltpu.async_copy(ed_hbm.at[crow + j], ibufs[j], isems[j])
                pltpu.async_copy(wt_hbm.at[crow + j], wbufs[j], wsems[j])
            for j in range(2):
                pltpu.make_async_copy(ed_hbm.at[crow + j], ibufs[j], isems[j]).wait()
            pltpu.async_copy(tab_hbm.at[i0.at[0]], g0, sg0)
            pltpu.async_copy(tab_hbm.at[i1.at[0]], g1, sg1)

            def quad(q, carry):
                for j in range(4):
                    ci = q * 4 + j
                    gb, sg = gbufs[j % 2]
                    ib, isem = ibufs[j], isems[j]
                    wb, wsem = wbufs[j], wsems[j]
                    pltpu.make_async_copy(tab_hbm.at[ib.at[0]], gb, sg).wait()
                    pltpu.make_async_copy(wt_hbm.at[crow + ci], wb, wsem).wait()

                    @pl.when(ci >= 1)
                    def _():
                        pltpu.make_async_copy(s0, acc_sh.at[ib.at[1]], ss0).wait()

                    def scale(g, cc):
                        w16 = wb[0, pl.ds(g * 16, 16)]
                        for el in range(16):
                            wv = w16[el]
                            e = g * 16 + el
                            for d in range(_HALF // 16):
                                sl = pl.ds(d * 16, 16)
                                s0[e, sl] = gb[e, sl] * wv
                        return cc

                    lax.fori_loop(0, _K // 16, scale, 0)
                    pltpu.async_copy(s0, acc_sh.at[ib.at[1]], ss0, add=True)

                    @pl.when(ci + 2 < CH)
                    def _():
                        ib2, is2 = ibufs[(j + 2) % 4], isems[(j + 2) % 4]
                        pltpu.make_async_copy(
                            ed_hbm.at[crow + ci + 2], ib2, is2).wait()
                        pltpu.async_copy(tab_hbm.at[ib2.at[0]], gb, sg)

                    @pl.when(ci + 4 < CH)
                    def _():
                        pltpu.async_copy(ed_hbm.at[crow + ci + 4], ib, isem)
                        pltpu.async_copy(wt_hbm.at[crow + ci + 4], wb, wsem)
                return carry

            lax.fori_loop(0, CH // 4, quad, 0)
            pltpu.make_async_copy(s0, acc_sh.at[i3.at[1]], ss0).wait()
            plsc.subcore_barrier()
            pltpu.sync_copy(acc_sh.at[pl.ds(s * stripe, stripe)],
                            out_hbm.at[pl.ds(s * stripe, stripe)])

        @pl.when(c == 0)
        def _():
            process(lo_hbm, out_lo)

        @pl.when(c == 1)
        def _():
            process(hi_hbm, out_hi)

    return agg


def _tail_body(lo_ref, hi_ref, b_ref, out_ref):
    h = jnp.concatenate([lo_ref[...], hi_ref[...]], axis=-1)
    h = _proj(_exp0(h))
    b = _proj(_exp0(b_ref[...]))
    h = _mobius(h, b)
    h = _proj(h)
    h = jnp.tanh(_log0(h))
    out_ref[...] = _proj(_exp0(h))


def _tail(lo, hi, bias_vec, N):
    D = 2 * _HALF
    bn = 2000
    return pl.pallas_call(
        _tail_body,
        grid=(N // bn,),
        in_specs=[
            pl.BlockSpec((bn, _HALF), lambda i: (i, 0)),
            pl.BlockSpec((bn, _HALF), lambda i: (i, 0)),
            pl.BlockSpec((1, D), lambda i: (0, 0)),
        ],
        out_specs=pl.BlockSpec((bn, D), lambda i: (i, 0)),
        out_shape=jax.ShapeDtypeStruct((N, D), jnp.float32),
    )(lo, hi, bias_vec)


def kernel(ents_embed_input, edge_index, edge_weight, W_ent, bias_vec):
    N, D = ents_embed_input.shape
    E = edge_index.shape[1]
    src = edge_index[0].astype(jnp.int32)
    dst = edge_index[1].astype(jnp.int32)
    w = edge_weight.astype(jnp.float32)

    CH = -(-(-(-E // (_NS * _K))) // 8) * 8
    EP = _NS * CH * _K
    if EP != E:
        pad_idx = jnp.arange(EP - E, dtype=jnp.int32) % N
        src = jnp.concatenate([src, pad_idx])
        dst = jnp.concatenate([dst, pad_idx])
        w = jnp.pad(w, (0, EP - E))
    R = _NS * CH
    ed = jnp.stack([src.reshape(R, _K), dst.reshape(R, _K)], axis=1)
    wt = w.reshape(R, 1, _K)

    lo, hi = _head(ents_embed_input, W_ent)
    NP = _NS * (-(-(-(-N // _NS)) // 8) * 8)
    zeros = jnp.zeros((NP, _HALF), jnp.float32)
    agg_lo, agg_hi = _make_agg(N, CH)(lo, hi, ed, wt, zeros)
    return _tail(agg_lo, agg_hi, bias_vec, N)


# speedup vs baseline: 7.2143x; 1.3700x over previous
import functools

import jax
import jax.numpy as jnp
from jax import lax
from jax.experimental import pallas as pl
from jax.experimental.pallas import tpu as pltpu
from jax.experimental.pallas import tpu_sc as plsc

_EPS = 1e-5
_MIN_NORM = 1e-10
_K = 96
_NS = 16
_HALF = 128


def _norm(x):
    return jnp.sqrt(jnp.clip(jnp.sum(x * x, axis=-1, keepdims=True), _MIN_NORM, None))


def _atanh(x):
    return 0.5 * jnp.log((1.0 + x) / (1.0 - x))


def _log0(x):
    n = _norm(x)
    nc = jnp.clip(n, _MIN_NORM, 1.0 - _EPS)
    return _atanh(nc) * x / n


def _exp0(v):
    n = _norm(v)
    return jnp.tanh(n) * v / n


def _proj(x):
    n = _norm(x)
    scale = jnp.minimum(jnp.ones_like(n), (1.0 - _EPS) / n)
    return x * scale


def _mobius(x, y):
    x2 = jnp.sum(x * x, axis=-1, keepdims=True)
    y2 = jnp.sum(y * y, axis=-1, keepdims=True)
    xy = jnp.sum(x * y, axis=-1, keepdims=True)
    num = (1.0 + 2.0 * xy + y2) * x + (1.0 - x2) * y
    den = 1.0 + 2.0 * xy + x2 * y2
    return num / (den + _MIN_NORM)


def _head_body(x_ref, w_ref, lo_ref, hi_ref):
    t = _log0(x_ref[...])
    m = jnp.dot(t, w_ref[...], preferred_element_type=jnp.float32)
    lo_ref[...] = m[:, :_HALF]
    hi_ref[...] = m[:, _HALF:]


def _head(x, W):
    N, D = x.shape
    bn = 2000
    return pl.pallas_call(
        _head_body,
        grid=(N // bn,),
        in_specs=[
            pl.BlockSpec((bn, D), lambda i: (i, 0)),
            pl.BlockSpec((D, D), lambda i: (0, 0)),
        ],
        out_specs=[
            pl.BlockSpec((bn, _HALF), lambda i: (i, 0)),
            pl.BlockSpec((bn, _HALF), lambda i: (i, 0)),
        ],
        out_shape=[
            jax.ShapeDtypeStruct((N, _HALF), jnp.float32),
            jax.ShapeDtypeStruct((N, _HALF), jnp.float32),
        ],
    )(x, W)


@functools.lru_cache(maxsize=None)
def _make_agg(N, CH):
    stripe = -(-(-(-N // _NS)) // 8) * 8
    NP = stripe * _NS
    mesh = plsc.VectorSubcoreMesh(core_axis_name="c", subcore_axis_name="s")

    @functools.partial(
        pl.kernel,
        mesh=mesh,
        out_type=[
            jax.ShapeDtypeStruct((NP, _HALF), jnp.float32),
            jax.ShapeDtypeStruct((NP, _HALF), jnp.float32),
        ],
        scratch_types=[
            pltpu.VMEM((4, 3, _K), jnp.int32),
            pltpu.VMEM((_K, _HALF), jnp.float32),
            pltpu.VMEM((_K, _HALF), jnp.float32),
            pltpu.VMEM((_K, _HALF), jnp.float32),
            pltpu.VMEM_SHARED((NP, _HALF), jnp.float32),
            pltpu.SemaphoreType.DMA,
            pltpu.SemaphoreType.DMA,
            pltpu.SemaphoreType.DMA,
            pltpu.SemaphoreType.DMA,
            pltpu.SemaphoreType.DMA,
            pltpu.SemaphoreType.DMA,
            pltpu.SemaphoreType.DMA,
        ],
    )
    def agg(lo_hbm, hi_hbm, ed_hbm, z_hbm,
            out_lo, out_hi, ed_ring,
            g0, g1, s0, acc_sh,
            si0, si1, si2, si3, sg0, sg1, ss0):
        c = lax.axis_index("c")
        s = lax.axis_index("s")
        isems = (si0, si1, si2, si3)
        gbufs = ((g0, sg0), (g1, sg1))

        def process(tab_hbm, out_hbm):
            pltpu.sync_copy(z_hbm.at[pl.ds(s * stripe, stripe)],
                            acc_sh.at[pl.ds(s * stripe, stripe)])
            plsc.subcore_barrier()
            crow = s * CH
            for j in range(4):
                pltpu.async_copy(ed_hbm.at[crow + j], ed_ring.at[j], isems[j])
            for j in range(2):
                pltpu.make_async_copy(
                    ed_hbm.at[crow + j], ed_ring.at[j], isems[j]).wait()
            pltpu.async_copy(tab_hbm.at[ed_ring.at[0, 0]], g0, sg0)
            pltpu.async_copy(tab_hbm.at[ed_ring.at[1, 0]], g1, sg1)

            def quad(q, carry):
                for j in range(4):
                    ci = q * 4 + j
                    gb, sg = gbufs[j % 2]
                    isem = isems[j]
                    pltpu.make_async_copy(
                        tab_hbm.at[ed_ring.at[j, 0]], gb, sg).wait()

                    @pl.when(ci >= 1)
                    def _():
                        pltpu.make_async_copy(
                            s0, acc_sh.at[ed_ring.at[j, 1]], ss0).wait()

                    def scale(g, cc):
                        w16i = ed_ring[j, 2, pl.ds(g * 16, 16)]
                        for el in range(16):
                            wv = jax.lax.bitcast_convert_type(
                                w16i[el], jnp.float32)
                            e = g * 16 + el
                            for d in range(_HALF // 16):
                                sl = pl.ds(d * 16, 16)
                                s0[e, sl] = gb[e, sl] * wv
                        return cc

                    lax.fori_loop(0, _K // 16, scale, 0, unroll=2)
                    pltpu.async_copy(s0, acc_sh.at[ed_ring.at[j, 1]], ss0,
                                     add=True)

                    @pl.when(ci + 2 < CH)
                    def _():
                        j2 = (j + 2) % 4
                        pltpu.make_async_copy(
                            ed_hbm.at[crow + ci + 2], ed_ring.at[j2],
                            isems[j2]).wait()
                        pltpu.async_copy(tab_hbm.at[ed_ring.at[j2, 0]], gb, sg)

                    @pl.when(ci + 4 < CH)
                    def _():
                        pltpu.async_copy(ed_hbm.at[crow + ci + 4],
                                         ed_ring.at[j], isem)
                return carry

            lax.fori_loop(0, CH // 4, quad, 0)
            pltpu.make_async_copy(s0, acc_sh.at[ed_ring.at[3, 1]], ss0).wait()
            plsc.subcore_barrier()
            pltpu.sync_copy(acc_sh.at[pl.ds(s * stripe, stripe)],
                            out_hbm.at[pl.ds(s * stripe, stripe)])

        @pl.when(c == 0)
        def _():
            process(lo_hbm, out_lo)

        @pl.when(c == 1)
        def _():
            process(hi_hbm, out_hi)

    return agg


def _tail_body(lo_ref, hi_ref, b_ref, out_ref):
    h = jnp.concatenate([lo_ref[...], hi_ref[...]], axis=-1)
    h = _proj(_exp0(h))
    b = _proj(_exp0(b_ref[...]))
    h = _mobius(h, b)
    h = _proj(h)
    h = jnp.tanh(_log0(h))
    out_ref[...] = _proj(_exp0(h))


def _tail(lo, hi, bias_vec, N):
    D = 2 * _HALF
    bn = 2000
    return pl.pallas_call(
        _tail_body,
        grid=(N // bn,),
        in_specs=[
            pl.BlockSpec((bn, _HALF), lambda i: (i, 0)),
            pl.BlockSpec((bn, _HALF), lambda i: (i, 0)),
            pl.BlockSpec((1, D), lambda i: (0, 0)),
        ],
        out_specs=pl.BlockSpec((bn, D), lambda i: (i, 0)),
        out_shape=jax.ShapeDtypeStruct((N, D), jnp.float32),
    )(lo, hi, bias_vec)


def kernel(ents_embed_input, edge_index, edge_weight, W_ent, bias_vec):
    N, D = ents_embed_input.shape
    E = edge_index.shape[1]
    src = edge_index[0].astype(jnp.int32)
    dst = edge_index[1].astype(jnp.int32)
    w = edge_weight.astype(jnp.float32)

    CH = -(-(-(-E // (_NS * _K))) // 4) * 4
    EP = _NS * CH * _K
    if EP != E:
        pad_idx = jnp.arange(EP - E, dtype=jnp.int32) % N
        src = jnp.concatenate([src, pad_idx])
        dst = jnp.concatenate([dst, pad_idx])
        w = jnp.pad(w, (0, EP - E))
    R = _NS * CH
    wi = jax.lax.bitcast_convert_type(w, jnp.int32)
    ed = jnp.stack(
        [src.reshape(R, _K), dst.reshape(R, _K), wi.reshape(R, _K)], axis=1)

    lo, hi = _head(ents_embed_input, W_ent)
    NP = _NS * (-(-(-(-N // _NS)) // 8) * 8)
    zeros = jnp.zeros((NP, _HALF), jnp.float32)
    agg_lo, agg_hi = _make_agg(N, CH)(lo, hi, ed, zeros)
    return _tail(agg_lo, agg_hi, bias_vec, N)
